# baseline (speedup 1.0000x reference)
import functools

import jax
import jax.numpy as jnp
from jax import lax
from jax.experimental import pallas as pl
from jax.experimental.pallas import tpu as pltpu
from jax.experimental.pallas import tpu_sc as plsc

BATCH = 16384
EMBED = 16
REG_SCALE = 1e-05 * 0.5

_INFO = plsc.get_sparse_core_info()
NC = _INFO.num_cores
NS = _INFO.num_subcores
NW = NC * NS
BPW = BATCH // NW
GROUPS = BPW // 16


def _sc_body(user_h, pos_h, neg_h, ue_h, ie_h,
             d_out, acc_out,
             idx_u, idx_p, idx_n,
             u_v, p_v, n_v, d_v, acc_v, sem):
    wid = lax.axis_index("s") * NC + lax.axis_index("c")
    base = wid * BPW

    pltpu.sync_copy(user_h.at[pl.ds(base, BPW)], idx_u)
    pltpu.sync_copy(pos_h.at[pl.ds(base, BPW)], idx_p)
    pltpu.sync_copy(neg_h.at[pl.ds(base, BPW)], idx_n)

    cu = pltpu.async_copy(ue_h.at[idx_u], u_v, sem)
    cp = pltpu.async_copy(ie_h.at[idx_p], p_v, sem)
    cn = pltpu.async_copy(ie_h.at[idx_n], n_v, sem)
    cu.wait()
    cp.wait()
    cn.wait()

    row0 = lax.iota(jnp.int32, 16)

    def group(g, acc):
        rows = g * 16 + row0
        dvec = jnp.zeros((16,), jnp.float32)
        for j in range(16):
            col = jnp.full((16,), j, jnp.int32)
            uc = plsc.load_gather(u_v, [rows, col])
            pc = plsc.load_gather(p_v, [rows, col])
            nc = plsc.load_gather(n_v, [rows, col])
            dvec = dvec + uc * (pc - nc)
            acc = acc + uc * uc + pc * pc + nc * nc
        d_v[pl.ds(g * 16, 16)] = dvec
        return acc

    acc = lax.fori_loop(0, GROUPS, group, jnp.zeros((16,), jnp.float32))
    acc_v[...] = acc

    pltpu.sync_copy(d_v, d_out.at[pl.ds(base, BPW)])
    pltpu.sync_copy(acc_v, acc_out.at[wid])


@functools.partial(
    pl.kernel,
    mesh=plsc.VectorSubcoreMesh(core_axis_name="c", subcore_axis_name="s"),
    compiler_params=pltpu.CompilerParams(
        needs_layout_passes=False, use_tc_tiling_on_sc=False),
    out_type=[
        jax.ShapeDtypeStruct((BATCH,), jnp.float32),
        jax.ShapeDtypeStruct((NW, EMBED), jnp.float32),
    ],
    scratch_types=[
        pltpu.VMEM((BPW,), jnp.int32),
        pltpu.VMEM((BPW,), jnp.int32),
        pltpu.VMEM((BPW,), jnp.int32),
        pltpu.VMEM((BPW, EMBED), jnp.float32),
        pltpu.VMEM((BPW, EMBED), jnp.float32),
        pltpu.VMEM((BPW, EMBED), jnp.float32),
        pltpu.VMEM((BPW,), jnp.float32),
        pltpu.VMEM((EMBED,), jnp.float32),
        pltpu.SemaphoreType.DMA,
    ],
)
def _sc_kernel(user_h, pos_h, neg_h, ue_h, ie_h, d_out, acc_out,
               idx_u, idx_p, idx_n, u_v, p_v, n_v, d_v, acc_v, sem):
    _sc_body(user_h, pos_h, neg_h, ue_h, ie_h, d_out, acc_out,
             idx_u, idx_p, idx_n, u_v, p_v, n_v, d_v, acc_v, sem)


def _tc_body(d_ref, acc_ref, bpr_ref, reg_ref):
    x = d_ref[...]
    s = jnp.log(jax.nn.sigmoid(x))
    bpr_ref[0, 0] = -jnp.sum(s) / jnp.float32(BATCH)
    reg_ref[0, 0] = jnp.float32(REG_SCALE) * jnp.sum(acc_ref[...])


_tc_finish = pl.pallas_call(
    _tc_body,
    out_shape=[
        jax.ShapeDtypeStruct((1, 1), jnp.float32),
        jax.ShapeDtypeStruct((1, 1), jnp.float32),
    ],
    in_specs=[
        pl.BlockSpec(memory_space=pltpu.VMEM),
        pl.BlockSpec(memory_space=pltpu.VMEM),
    ],
    out_specs=[
        pl.BlockSpec(memory_space=pltpu.SMEM),
        pl.BlockSpec(memory_space=pltpu.SMEM),
    ],
)


def kernel(user, pos, neg, user_embedding, item_embedding):
    user = user.astype(jnp.int32)
    pos = pos.astype(jnp.int32)
    neg = neg.astype(jnp.int32)
    d, acc = _sc_kernel(user, pos, neg, user_embedding, item_embedding)
    bpr, reg = _tc_finish(d.reshape(128, 128), acc)
    return (bpr[0, 0], reg[0, 0])
